# baseline (speedup 1.0000x reference)
import functools

import jax
import jax.numpy as jnp
from jax import lax
from jax.experimental import pallas as pl
from jax.experimental.pallas import tpu as pltpu
from jax.experimental.pallas import tpu_sc as plsc

B = 16384
D = 32

_INFO = plsc.get_sparse_core_info()
_NC = _INFO.num_cores
_NS = _INFO.num_subcores
_NW = _NC * _NS
_BPW = B // _NW
_CHUNK = 128
_NCHUNK = _BPW // _CHUNK


def _body(subj2d, rel2d, obj2d, entity_hbm, relation_hbm, out_hbm,
          sidx_v, ridx_v, oidx_v, e1_v, r_v, e2_v, out_v, sem, idx_sem):
    wid = lax.axis_index("c") * _NS + lax.axis_index("s")
    base = wid * _BPW

    row0 = wid * _NCHUNK
    pltpu.async_copy(subj2d.at[pl.ds(row0, _NCHUNK)], sidx_v, idx_sem).wait()
    pltpu.async_copy(rel2d.at[pl.ds(row0, _NCHUNK)], ridx_v, idx_sem).wait()
    pltpu.async_copy(obj2d.at[pl.ds(row0, _NCHUNK)], oidx_v, idx_sem).wait()

    descs = []
    for j in range(_NCHUNK):
        dst = pl.ds(j * _CHUNK, _CHUNK)
        descs.append(pltpu.async_copy(entity_hbm.at[sidx_v.at[j]], e1_v.at[dst], sem))
        descs.append(pltpu.async_copy(relation_hbm.at[ridx_v.at[j]], r_v.at[dst], sem))
        descs.append(pltpu.async_copy(entity_hbm.at[oidx_v.at[j]], e2_v.at[dst], sem))
    for dsc in descs:
        dsc.wait()

    lane = lax.iota(jnp.int32, 16)
    bitrev = (((lane & 1) << 3) | ((lane & 2) << 1)
              | ((lane & 4) >> 1) | ((lane & 8) >> 3))

    _dnums = lax.GatherDimensionNumbers(
        offset_dims=(), collapsed_slice_dims=(0,), start_index_map=(0,))

    def shuf(v, idx):
        return lax.gather(v, idx[:, None], _dnums, (1,),
                          mode=lax.GatherScatterMode.PROMISE_IN_BOUNDS)

    def group(g, _):
        vecs = []
        for u in range(16):
            r = g * 16 + u
            vecs.append(
                e1_v[r, pl.ds(0, 16)] * r_v[r, pl.ds(0, 16)] * e2_v[r, pl.ds(0, 16)]
                + e1_v[r, pl.ds(16, 16)] * r_v[r, pl.ds(16, 16)] * e2_v[r, pl.ds(16, 16)])
        for k in (8, 4, 2, 1):
            m = (lane & k) == 0
            idx = lane ^ k
            vecs = [jnp.where(m, a + shuf(a, idx), b + shuf(b, idx))
                    for a, b in zip(vecs[0::2], vecs[1::2])]
        out_v[pl.ds(g * 16, 16)] = shuf(vecs[0], bitrev)
        return 0

    lax.fori_loop(0, _BPW // 16, group, 0, unroll=False)

    pltpu.async_copy(out_v, out_hbm.at[pl.ds(base, _BPW)], idx_sem).wait()


@jax.jit
def _run(entity_table, relation_table, subj2d, rel2d, obj2d):
    mesh = plsc.VectorSubcoreMesh(core_axis_name="c", subcore_axis_name="s")
    kfn = pl.kernel(
        functools.partial(_body),
        out_type=jax.ShapeDtypeStruct((B,), jnp.float32),
        mesh=mesh,
        compiler_params=pltpu.CompilerParams(use_tc_tiling_on_sc=False),
        scratch_types=[
            pltpu.VMEM((_NCHUNK, _CHUNK), jnp.int32),
            pltpu.VMEM((_NCHUNK, _CHUNK), jnp.int32),
            pltpu.VMEM((_NCHUNK, _CHUNK), jnp.int32),
            pltpu.VMEM((_BPW, D), jnp.float32),
            pltpu.VMEM((_BPW, D), jnp.float32),
            pltpu.VMEM((_BPW, D), jnp.float32),
            pltpu.VMEM((_BPW,), jnp.float32),
            pltpu.SemaphoreType.DMA,
            pltpu.SemaphoreType.DMA,
        ],
    )
    return kfn(subj2d, rel2d, obj2d, entity_table, relation_table)


def kernel(entity_table, relation_table, subj_idx, rel_idx, obj_idx):
    subj2d = subj_idx.astype(jnp.int32).reshape(_NW * _NCHUNK, _CHUNK)
    rel2d = rel_idx.astype(jnp.int32).reshape(_NW * _NCHUNK, _CHUNK)
    obj2d = obj_idx.astype(jnp.int32).reshape(_NW * _NCHUNK, _CHUNK)
    return _run(entity_table, relation_table, subj2d, rel2d, obj2d)
